# baseline (speedup 1.0000x reference)
import functools

import jax
import jax.numpy as jnp
from jax import lax
from jax.experimental import pallas as pl
from jax.experimental.pallas import tpu as pltpu
from jax.experimental.pallas import tpu_sc as plsc

N_NODES = 10000
D_IN = 128
D_HID = 128
D_OUT = 2
N_EDGES = 320000

NC = 2
NS = 16
NW = NC * NS
CH = 125
EW = N_EDGES // NW
NCHUNK = EW // CH
NPAD = 10240
ROWS_PT = NPAD // NS
SLAB = 128
RSLABS = ROWS_PT // SLAB
DEG_PT = NPAD // NS
DH = D_HID // NC
EW2 = N_EDGES // NS
NCHUNK2 = EW2 // CH
NB = 5
GD = 3

_mesh = plsc.VectorSubcoreMesh(
    core_axis_name="c", subcore_axis_name="s", num_cores=NC, num_subcores=NS)


def _wid(c, s):
    return s * NC + c


@functools.partial(
    pl.kernel,
    out_type=jax.ShapeDtypeStruct((NC, NPAD), jnp.float32),
    mesh=_mesh,
    scratch_types=[
        pltpu.VMEM((NCHUNK, CH), jnp.int32),
        pltpu.VMEM((128,), jnp.float32),
        pltpu.VMEM((DEG_PT,), jnp.float32),
        pltpu.VMEM_SHARED((NPAD,), jnp.float32),
    ],
)
def _deg_kernel(dst_hbm, out_hbm, idx_v, ones_v, buf_v, acc):
    c = lax.axis_index("c")
    s = lax.axis_index("s")
    w = _wid(c, s)

    for i in range(128 // 16):
        ones_v[pl.ds(i * 16, 16)] = jnp.ones((16,), jnp.float32)
    for i in range(DEG_PT // 16):
        buf_v[pl.ds(i * 16, 16)] = jnp.zeros((16,), jnp.float32)
    pltpu.sync_copy(buf_v, acc.at[pl.ds(s * DEG_PT, DEG_PT)])
    pltpu.sync_copy(dst_hbm.at[pl.ds(w * NCHUNK, NCHUNK)], idx_v)
    plsc.subcore_barrier()

    def body(g, carry):
        pltpu.sync_copy(ones_v.at[pl.ds(0, CH)], acc.at[idx_v.at[g]], add=True)
        return carry

    lax.fori_loop(0, NCHUNK, body, 0, unroll=False)
    plsc.subcore_barrier()
    pltpu.sync_copy(acc.at[pl.ds(s * DEG_PT, DEG_PT)], buf_v)
    pltpu.sync_copy(buf_v, out_hbm.at[c, pl.ds(s * DEG_PT, DEG_PT)])


@functools.partial(
    pl.kernel,
    out_type=jax.ShapeDtypeStruct((NPAD, D_HID), jnp.float32),
    mesh=_mesh,
    scratch_types=[
        pltpu.VMEM((NCHUNK2, CH), jnp.int32),
        pltpu.VMEM((NCHUNK2, CH), jnp.int32),
        pltpu.VMEM((NB, CH, DH), jnp.float32),
        pltpu.VMEM((SLAB, DH), jnp.float32),
        pltpu.VMEM_SHARED((NPAD, DH), jnp.float32),
        pltpu.SemaphoreType.DMA((NB,)),
        pltpu.SemaphoreType.DMA((NB,)),
    ],
    compiler_params=pltpu.CompilerParams(use_tc_tiling_on_sc=False),
)
def _agg_kernel(hp_hbm, src_hbm, dst_hbm, out_hbm, src_v, dst_v, rows_v,
                slab_v, acc, gsem, ssem):
    c = lax.axis_index("c")
    s = lax.axis_index("s")

    def zrow(i, carry):
        for j in range(DH // 16):
            slab_v[i, pl.ds(j * 16, 16)] = jnp.zeros((16,), jnp.float32)
        return carry

    lax.fori_loop(0, SLAB, zrow, 0, unroll=False)
    for i in range(RSLABS):
        pltpu.sync_copy(slab_v, acc.at[pl.ds(s * ROWS_PT + i * SLAB, SLAB)])
    pltpu.sync_copy(src_hbm.at[c, pl.ds(s * NCHUNK2, NCHUNK2)], src_v)
    pltpu.sync_copy(dst_hbm.at[pl.ds(s * NCHUNK2, NCHUNK2)], dst_v)
    plsc.subcore_barrier()

    def gather_start(g, b):
        pltpu.async_copy(hp_hbm.at[src_v.at[g]], rows_v.at[b], gsem.at[b])

    def gather_wait(g, b):
        pltpu.make_async_copy(
            hp_hbm.at[src_v.at[g]], rows_v.at[b], gsem.at[b]).wait()

    def scatter_start(g, b):
        pltpu.async_copy(rows_v.at[b], acc.at[dst_v.at[g]], ssem.at[b],
                         add=True)

    def scatter_wait(g, b):
        pltpu.make_async_copy(
            rows_v.at[b], acc.at[dst_v.at[g]], ssem.at[b]).wait()

    for b in range(GD):
        gather_start(b, b)

    def body(t, carry):
        for j in range(NB):
            g = t * NB + j
            bn = (j + GD) % NB

            @pl.when(g + GD - NB >= 0)
            def _():
                scatter_wait(g + GD - NB, bn)

            @pl.when(g + GD < NCHUNK2)
            def _():
                gather_start(g + GD, bn)

            gather_wait(g, j)
            scatter_start(g, j)
        return carry

    lax.fori_loop(0, NCHUNK2 // NB, body, 0, unroll=False)
    for k in range(NB - GD):
        gtail = NCHUNK2 - (NB - GD) + k
        scatter_wait(gtail, gtail % NB)
    plsc.subcore_barrier()
    for i in range(RSLABS):
        r0 = s * ROWS_PT + i * SLAB
        pltpu.sync_copy(acc.at[pl.ds(r0, SLAB)], slab_v)
        pltpu.sync_copy(slab_v,
                        out_hbm.at[pl.ds(r0, SLAB), pl.ds(c * DH, DH)])


def _mm_body(x_ref, w1_ref, h_ref):
    h_ref[...] = jnp.dot(x_ref[...], w1_ref[...],
                         preferred_element_type=jnp.float32)


def _scale_body(degp_ref, h_ref, hp_ref, dis_ref):
    deg = degp_ref[0, :N_NODES] + degp_ref[1, :N_NODES] + 1.0
    dis = lax.rsqrt(deg)[:, None]
    dis_ref[...] = dis
    hp_ref[...] = h_ref[...] * dis


def _combine(p_ref, hp_ref, dis_ref, b_ref):
    z = dis_ref[...] * (p_ref[:N_NODES] + hp_ref[...]) + b_ref[...]
    return jnp.maximum(z, 0.0)


def _tc2_body(p_ref, hp_ref, dis_ref, b_ref, w2_ref, out_ref):
    z = _combine(p_ref, hp_ref, dis_ref, b_ref)
    h = jnp.dot(z, w2_ref[...], preferred_element_type=jnp.float32)
    out_ref[...] = h * dis_ref[...]


def _tc3_body(p_ref, hp_ref, dis_ref, b_ref, wfc_ref, bfc_ref, out_ref):
    z = _combine(p_ref, hp_ref, dis_ref, b_ref)
    out_ref[...] = (
        jnp.dot(z, wfc_ref[...], preferred_element_type=jnp.float32)
        + bfc_ref[...]
    )


def kernel(x, edge_index, W1, b1, W2, b2, Wfc, bfc):
    src = edge_index[0].reshape(N_EDGES // CH, CH)
    dst = edge_index[1].reshape(N_EDGES // CH, CH)
    srcg = jnp.stack((2 * src, 2 * src + 1))

    degp = _deg_kernel(dst)

    h1 = pl.pallas_call(
        _mm_body,
        out_shape=jax.ShapeDtypeStruct((N_NODES, D_HID), jnp.float32),
    )(x, W1)

    hp1, dis = pl.pallas_call(
        _scale_body,
        out_shape=(
            jax.ShapeDtypeStruct((N_NODES, D_HID), jnp.float32),
            jax.ShapeDtypeStruct((N_NODES, 1), jnp.float32),
        ),
    )(degp, h1)

    hp1_flat = lax.optimization_barrier(hp1.reshape(NC * N_NODES * DH))
    p1 = _agg_kernel(hp1_flat.reshape(NC * N_NODES, DH), srcg, dst)

    hp2 = pl.pallas_call(
        _tc2_body,
        out_shape=jax.ShapeDtypeStruct((N_NODES, D_HID), jnp.float32),
    )(p1, hp1, dis, b1.reshape(1, D_HID), W2)

    hp2_flat = lax.optimization_barrier(hp2.reshape(NC * N_NODES * DH))
    p2 = _agg_kernel(hp2_flat.reshape(NC * N_NODES, DH), srcg, dst)

    out = pl.pallas_call(
        _tc3_body,
        out_shape=jax.ShapeDtypeStruct((N_NODES, D_OUT), jnp.float32),
    )(p2, hp2, dis, b2.reshape(1, D_HID), Wfc, bfc.reshape(1, D_OUT))

    return out


# speedup vs baseline: 1.0231x; 1.0231x over previous
import functools

import jax
import jax.numpy as jnp
from jax import lax
from jax.experimental import pallas as pl
from jax.experimental.pallas import tpu as pltpu
from jax.experimental.pallas import tpu_sc as plsc

N_NODES = 10000
D_IN = 128
D_HID = 128
D_OUT = 2
N_EDGES = 320000

NC = 2
NS = 16
NW = NC * NS
CH = 125
EW = N_EDGES // NW
NCHUNK = EW // CH
NPAD = 10240
ROWS_PT = NPAD // NS
SLAB = 128
RSLABS = ROWS_PT // SLAB
DEG_PT = NPAD // NS
DH = D_HID // NC
EW2 = N_EDGES // NS
NCHUNK2 = EW2 // CH
NB = 5
GD = 3

_mesh = plsc.VectorSubcoreMesh(
    core_axis_name="c", subcore_axis_name="s", num_cores=NC, num_subcores=NS)


def _wid(c, s):
    return s * NC + c


@functools.partial(
    pl.kernel,
    out_type=jax.ShapeDtypeStruct((NC, NPAD), jnp.float32),
    mesh=_mesh,
    scratch_types=[
        pltpu.VMEM((NCHUNK, CH), jnp.int32),
        pltpu.VMEM((128,), jnp.float32),
        pltpu.VMEM((DEG_PT,), jnp.float32),
        pltpu.VMEM_SHARED((NPAD,), jnp.float32),
        pltpu.SemaphoreType.DMA((8,)),
    ],
)
def _deg_kernel(dst_hbm, out_hbm, idx_v, ones_v, buf_v, acc, dsem):
    c = lax.axis_index("c")
    s = lax.axis_index("s")
    w = _wid(c, s)

    for i in range(128 // 16):
        ones_v[pl.ds(i * 16, 16)] = jnp.ones((16,), jnp.float32)
    for i in range(DEG_PT // 16):
        buf_v[pl.ds(i * 16, 16)] = jnp.zeros((16,), jnp.float32)
    pltpu.sync_copy(buf_v, acc.at[pl.ds(s * DEG_PT, DEG_PT)])
    pltpu.sync_copy(dst_hbm.at[pl.ds(w * NCHUNK, NCHUNK)], idx_v)
    plsc.subcore_barrier()

    def dwait(g):
        pltpu.make_async_copy(
            ones_v.at[pl.ds(0, CH)], acc.at[idx_v.at[g]], dsem.at[g % 8]
        ).wait()

    def body(g, carry):
        @pl.when(g >= 8)
        def _():
            dwait(g - 8)

        pltpu.async_copy(ones_v.at[pl.ds(0, CH)], acc.at[idx_v.at[g]],
                         dsem.at[g % 8], add=True)
        return carry

    lax.fori_loop(0, NCHUNK, body, 0, unroll=False)
    for k in range(8):
        dwait(NCHUNK - 8 + k)
    plsc.subcore_barrier()
    pltpu.sync_copy(acc.at[pl.ds(s * DEG_PT, DEG_PT)], buf_v)
    pltpu.sync_copy(buf_v, out_hbm.at[c, pl.ds(s * DEG_PT, DEG_PT)])


@functools.partial(
    pl.kernel,
    out_type=jax.ShapeDtypeStruct((NPAD, D_HID), jnp.float32),
    mesh=_mesh,
    scratch_types=[
        pltpu.VMEM((NCHUNK2, CH), jnp.int32),
        pltpu.VMEM((NCHUNK2, CH), jnp.int32),
        pltpu.VMEM((NB, CH, DH), jnp.float32),
        pltpu.VMEM((SLAB, DH), jnp.float32),
        pltpu.VMEM_SHARED((NPAD, DH), jnp.float32),
        pltpu.SemaphoreType.DMA((NB,)),
        pltpu.SemaphoreType.DMA((NB,)),
    ],
    compiler_params=pltpu.CompilerParams(use_tc_tiling_on_sc=False),
)
def _agg_kernel(hp_hbm, src_hbm, dst_hbm, out_hbm, src_v, dst_v, rows_v,
                slab_v, acc, gsem, ssem):
    c = lax.axis_index("c")
    s = lax.axis_index("s")

    def zrow(i, carry):
        for j in range(DH // 16):
            slab_v[i, pl.ds(j * 16, 16)] = jnp.zeros((16,), jnp.float32)
        return carry

    lax.fori_loop(0, SLAB, zrow, 0, unroll=False)
    for i in range(RSLABS):
        pltpu.sync_copy(slab_v, acc.at[pl.ds(s * ROWS_PT + i * SLAB, SLAB)])
    pltpu.sync_copy(src_hbm.at[c, pl.ds(s * NCHUNK2, NCHUNK2)], src_v)
    pltpu.sync_copy(dst_hbm.at[pl.ds(s * NCHUNK2, NCHUNK2)], dst_v)
    plsc.subcore_barrier()

    def gather_start(g, b):
        pltpu.async_copy(hp_hbm.at[src_v.at[g]], rows_v.at[b], gsem.at[b])

    def gather_wait(g, b):
        pltpu.make_async_copy(
            hp_hbm.at[src_v.at[g]], rows_v.at[b], gsem.at[b]).wait()

    def scatter_start(g, b):
        pltpu.async_copy(rows_v.at[b], acc.at[dst_v.at[g]], ssem.at[b],
                         add=True)

    def scatter_wait(g, b):
        pltpu.make_async_copy(
            rows_v.at[b], acc.at[dst_v.at[g]], ssem.at[b]).wait()

    for b in range(GD):
        gather_start(b, b)

    def body(t, carry):
        for j in range(NB):
            g = t * NB + j
            bn = (j + GD) % NB

            @pl.when(g + GD - NB >= 0)
            def _():
                scatter_wait(g + GD - NB, bn)

            @pl.when(g + GD < NCHUNK2)
            def _():
                gather_start(g + GD, bn)

            gather_wait(g, j)
            scatter_start(g, j)
        return carry

    lax.fori_loop(0, NCHUNK2 // NB, body, 0, unroll=False)
    for k in range(NB - GD):
        gtail = NCHUNK2 - (NB - GD) + k
        scatter_wait(gtail, gtail % NB)
    plsc.subcore_barrier()
    for i in range(RSLABS):
        r0 = s * ROWS_PT + i * SLAB
        pltpu.sync_copy(acc.at[pl.ds(r0, SLAB)], slab_v)
        pltpu.sync_copy(slab_v,
                        out_hbm.at[pl.ds(r0, SLAB), pl.ds(c * DH, DH)])


def _tc1_body(degp_ref, x_ref, w1_ref, hp_ref, dis_ref):
    deg = degp_ref[0, :N_NODES] + degp_ref[1, :N_NODES] + 1.0
    dis = lax.rsqrt(deg)[:, None]
    dis_ref[...] = dis
    h = jnp.dot(x_ref[...], w1_ref[...], preferred_element_type=jnp.float32)
    hp_ref[...] = h * dis


def _combine(p_ref, hp_ref, dis_ref, b_ref):
    z = dis_ref[...] * (p_ref[:N_NODES] + hp_ref[...]) + b_ref[...]
    return jnp.maximum(z, 0.0)


def _tc2_body(p_ref, hp_ref, dis_ref, b_ref, w2_ref, out_ref):
    z = _combine(p_ref, hp_ref, dis_ref, b_ref)
    h = jnp.dot(z, w2_ref[...], preferred_element_type=jnp.float32)
    out_ref[...] = h * dis_ref[...]


def _tc3_body(p_ref, hp_ref, dis_ref, b_ref, wfc_ref, bfc_ref, out_ref):
    z = _combine(p_ref, hp_ref, dis_ref, b_ref)
    out_ref[...] = (
        jnp.dot(z, wfc_ref[...], preferred_element_type=jnp.float32)
        + bfc_ref[...]
    )


def kernel(x, edge_index, W1, b1, W2, b2, Wfc, bfc):
    src = edge_index[0].reshape(N_EDGES // CH, CH)
    dst = edge_index[1].reshape(N_EDGES // CH, CH)
    srcg = jnp.stack((2 * src, 2 * src + 1))

    degp = _deg_kernel(dst)

    hp1, dis = pl.pallas_call(
        _tc1_body,
        out_shape=(
            jax.ShapeDtypeStruct((N_NODES, D_HID), jnp.float32),
            jax.ShapeDtypeStruct((N_NODES, 1), jnp.float32),
        ),
    )(degp, x, W1)

    hp1_flat = lax.optimization_barrier(hp1.reshape(NC * N_NODES * DH))
    p1 = _agg_kernel(hp1_flat.reshape(NC * N_NODES, DH), srcg, dst)

    hp2 = pl.pallas_call(
        _tc2_body,
        out_shape=jax.ShapeDtypeStruct((N_NODES, D_HID), jnp.float32),
    )(p1, hp1, dis, b1.reshape(1, D_HID), W2)

    hp2_flat = lax.optimization_barrier(hp2.reshape(NC * N_NODES * DH))
    p2 = _agg_kernel(hp2_flat.reshape(NC * N_NODES, DH), srcg, dst)

    out = pl.pallas_call(
        _tc3_body,
        out_shape=jax.ShapeDtypeStruct((N_NODES, D_OUT), jnp.float32),
    )(p2, hp2, dis, b2.reshape(1, D_HID), Wfc, bfc.reshape(1, D_OUT))

    return out
